# speedup vs baseline: 1.8368x; 1.8368x over previous
import jax
import jax.numpy as jnp
from jax import lax
from jax.experimental import pallas as pl

_N = 1024
_ROWS_PER_BLOCK = 256


def _bitonic_body(x_ref, o_ref):
    a = x_ref[...]
    col = lax.broadcasted_iota(jnp.int32, (1, _N), 1)
    k = 2
    while k <= _N:
        j = k // 2
        while j >= 1:
            low = (col & j) == 0
            asc = (col & k) == 0
            take_min = asc == low
            p = jnp.where(low, jnp.roll(a, -j, axis=1), jnp.roll(a, j, axis=1))
            mn = jnp.minimum(a, p)
            mx = jnp.maximum(a, p)
            a = jnp.where(take_min, mn, mx)
            j //= 2
        k *= 2
    o_ref[...] = a


def kernel(x):
    b, t, n = x.shape
    rows = b * t
    x2 = x.reshape(rows, n)
    grid = rows // _ROWS_PER_BLOCK
    out = pl.pallas_call(
        _bitonic_body,
        out_shape=jax.ShapeDtypeStruct((rows, n), x.dtype),
        grid=(grid,),
        in_specs=[pl.BlockSpec((_ROWS_PER_BLOCK, n), lambda g: (g, 0))],
        out_specs=pl.BlockSpec((_ROWS_PER_BLOCK, n), lambda g: (g, 0)),
    )(x2)
    return out.reshape(b, t, n)


# speedup vs baseline: 1.8695x; 1.0178x over previous
import jax
import jax.numpy as jnp
from jax import lax
from jax.experimental import pallas as pl

_N = 1024
_C = 128
_NCHUNK = _N // _C
_ROWS_PER_BLOCK = 256


def _bitonic_body(x_ref, o_ref):
    chunks = [x_ref[:, v * _C:(v + 1) * _C] for v in range(_NCHUNK)]
    lanes = lax.broadcasted_iota(jnp.int32, (1, _C), 1)

    k = 2
    while k <= _N:
        j = k // 2
        while j >= 1:
            if j >= _C:
                jc = j // _C
                for v in range(_NCHUNK):
                    if v & jc:
                        continue
                    w = v | jc
                    if k == _N:
                        asc = True
                    else:
                        asc = (v & (k // _C)) == 0
                    mn = jnp.minimum(chunks[v], chunks[w])
                    mx = jnp.maximum(chunks[v], chunks[w])
                    if asc:
                        chunks[v], chunks[w] = mn, mx
                    else:
                        chunks[v], chunks[w] = mx, mn
            else:
                low = (lanes & j) == 0
                if k <= _C // 2:
                    tm_asc = ((lanes & k) == 0) == low
                    tm_desc = None
                for v in range(_NCHUNK):
                    c = chunks[v]
                    p = jnp.where(low, jnp.roll(c, -j, axis=1),
                                  jnp.roll(c, j, axis=1))
                    mn = jnp.minimum(c, p)
                    mx = jnp.maximum(c, p)
                    if k <= _C // 2:
                        tm = tm_asc
                    else:
                        if k == _N:
                            asc = True
                        elif k == _C:
                            asc = (v & 1) == 0
                        else:
                            asc = (v & (k // _C)) == 0
                        tm = low if asc else ~low
                    chunks[v] = jnp.where(tm, mn, mx)
            j //= 2
        k *= 2

    for v in range(_NCHUNK):
        o_ref[:, v * _C:(v + 1) * _C] = chunks[v]


def kernel(x):
    b, t, n = x.shape
    rows = b * t
    x2 = x.reshape(rows, n)
    grid = rows // _ROWS_PER_BLOCK
    out = pl.pallas_call(
        _bitonic_body,
        out_shape=jax.ShapeDtypeStruct((rows, n), x.dtype),
        grid=(grid,),
        in_specs=[pl.BlockSpec((_ROWS_PER_BLOCK, n), lambda g: (g, 0))],
        out_specs=pl.BlockSpec((_ROWS_PER_BLOCK, n), lambda g: (g, 0)),
    )(x2)
    return out.reshape(b, t, n)


# speedup vs baseline: 2.4199x; 1.2944x over previous
import jax
import jax.numpy as jnp
from jax import lax
from jax.experimental import pallas as pl

_N = 1024
_C = 128
_NCHUNK = _N // _C
_ROWS_PER_BLOCK = 256


def _bitonic_body(x_ref, o_ref):
    chunks = [x_ref[:, v * _C:(v + 1) * _C] for v in range(_NCHUNK)]
    lanes = lax.broadcasted_iota(jnp.int32, (1, _C), 1)

    k = 2
    while k <= _N:
        j = k // 2
        while j >= 1:
            if j < _NCHUNK:
                if k < _NCHUNK:
                    asc_mask = None
                elif k < _N:
                    asc_mask = (lanes & (k >> 3)) == 0
                else:
                    asc_mask = None
                for v in range(_NCHUNK):
                    if v & j:
                        continue
                    w = v | j
                    mn = jnp.minimum(chunks[v], chunks[w])
                    mx = jnp.maximum(chunks[v], chunks[w])
                    if asc_mask is None:
                        asc = True if k == _N else (v & k) == 0
                        if asc:
                            chunks[v], chunks[w] = mn, mx
                        else:
                            chunks[v], chunks[w] = mx, mn
                    else:
                        chunks[v] = jnp.where(asc_mask, mn, mx)
                        chunks[w] = jnp.where(asc_mask, mx, mn)
            else:
                d = j // _NCHUNK
                low = (lanes & d) == 0
                if k == _N:
                    tm = low
                else:
                    tm = ((lanes & (k >> 3)) == 0) == low
                for v in range(_NCHUNK):
                    c = chunks[v]
                    p = jnp.where(low, jnp.roll(c, -d, axis=1),
                                  jnp.roll(c, d, axis=1))
                    mn = jnp.minimum(c, p)
                    mx = jnp.maximum(c, p)
                    chunks[v] = jnp.where(tm, mn, mx)
            j //= 2
        k *= 2

    for v in range(_NCHUNK):
        o_ref[:, v * _C:(v + 1) * _C] = chunks[v]


def kernel(x):
    b, t, n = x.shape
    rows = b * t
    x2 = x.reshape(rows, n)
    grid = rows // _ROWS_PER_BLOCK
    out = pl.pallas_call(
        _bitonic_body,
        out_shape=jax.ShapeDtypeStruct((rows, n), x.dtype),
        grid=(grid,),
        in_specs=[pl.BlockSpec((_ROWS_PER_BLOCK, n), lambda g: (g, 0))],
        out_specs=pl.BlockSpec((_ROWS_PER_BLOCK, n), lambda g: (g, 0)),
    )(x2)
    cols = jnp.arange(n, dtype=jnp.int32)
    perm = (cols & (_NCHUNK - 1)) * _C + (cols >> 3)
    return out[:, perm].reshape(b, t, n)
